# Initial kernel scaffold; baseline (speedup 1.0000x reference)
import jax
import jax.numpy as jnp
from jax.experimental import pallas as pl


def kernel(fore_map, fore_rep_encoded, target_embed, align_attns):
    raise NotImplementedError("write your pallas kernel here")


# speedup vs baseline: 2.2885x; 2.2885x over previous
import functools

import jax
import jax.numpy as jnp
from jax import lax
from jax.experimental import pallas as pl
from jax.experimental.pallas import tpu as pltpu
from jax.experimental.pallas import tpu_sc as plsc

_EPS = 1e-8
_NC = 2
_NS = 16
_LN = 16


def _select_body(k, kpad, te_ref, fr_ref, idx_ref, ws_ref):
    B, T, D = te_ref.shape
    for b in range(B):
        te = te_ref[b]
        fr = fr_ref[b]
        fr2 = fr[None, :]
        dot = jnp.sum(te * fr2, axis=1)
        n1 = jnp.maximum(jnp.sqrt(jnp.sum(te * te, axis=1)), _EPS)
        n2 = jnp.maximum(jnp.sqrt(jnp.sum(fr * fr)), _EPS)
        w = dot / (n1 * n2)
        wrow = w[None, :]
        wcol = w[:, None]

        C = 128
        rank = jnp.zeros((T,), jnp.int32)
        tt = lax.broadcasted_iota(jnp.int32, (C, T), 1)
        for jc in range(T // C):
            wc = wcol[jc * C:(jc + 1) * C]
            gt = (wc > wrow).astype(jnp.int32)
            jj = lax.broadcasted_iota(jnp.int32, (C, T), 0) + jc * C
            eq = ((wc == wrow) & (jj < tt)).astype(jnp.int32)
            rank = rank + jnp.sum(gt + eq, axis=0)

        rrow = rank[None, :]
        pcol = lax.broadcasted_iota(jnp.int32, (kpad, T), 0)
        tcand = lax.broadcasted_iota(jnp.int32, (kpad, T), 1)
        sel = (rrow == pcol) & (pcol < k)
        idx_ref[b, :] = jnp.sum(jnp.where(sel, tcand, 0), axis=1)
        ws_ref[b, :] = jnp.sum(jnp.where(sel, jnp.broadcast_to(wrow, (kpad, T)),
                                         0.0), axis=1)


def _select(target_embed, fore_rep, k, kpad):
    B, T, D = target_embed.shape
    return pl.pallas_call(
        functools.partial(_select_body, k, kpad),
        out_shape=(
            jax.ShapeDtypeStruct((B, kpad), jnp.int32),
            jax.ShapeDtypeStruct((B, kpad), jnp.float32),
        ),
    )(target_embed, fore_rep)


def _splat_minmax(x_mn, x_mx, buf):
    lanes = lax.iota(jnp.int32, 16)
    for shift in (8, 4, 2, 1):
        perm = lanes ^ shift
        buf[:] = x_mn
        x_mn = jnp.minimum(x_mn, plsc.load_gather(buf, [perm]))
        buf[:] = x_mx
        x_mx = jnp.maximum(x_mx, plsc.load_gather(buf, [perm]))
    return x_mn, x_mx


def _sc_total_body(H, S, PAIRS, attn_hbm, rows_hbm, w_hbm, out_hbm,
                   idx_v, wv, rbuf0, rbuf1, vrow, acc, shared, mbuf, mrow,
                   lbuf, sem0, sem1):
    c = lax.axis_index("c")
    s = lax.axis_index("s")
    wid = c * _NS + s

    pltpu.sync_copy(rows_hbm.at[wid], idx_v)
    pltpu.sync_copy(w_hbm.at[wid], wv)

    nchunk = S // _LN
    zero = jnp.zeros((_LN,), jnp.float32)

    def zstep(i, carry):
        acc[pl.ds(i * _LN, _LN)] = zero
        return carry
    lax.fori_loop(0, nchunk, zstep, 0)

    rbufs = (rbuf0, rbuf1)
    sems = (sem0, sem1)

    def start(j):
        buf = j % 2
        return pltpu.async_copy(attn_hbm.at[idx_v.at[j, pl.ds(0, H)]],
                                rbufs[buf], sems[buf])

    inv_h = jnp.float32(1.0 / H)
    cur = start(0)
    for j in range(PAIRS):
        nxt = start(j + 1) if j + 1 < PAIRS else None
        cur.wait()
        rb = rbufs[j % 2]
        wj = wv[j, :]

        def pass1(i, carry):
            mn, mx = carry
            col = pl.ds(i * _LN, _LN)
            ssum = rb[0, col]
            for h in range(1, H):
                ssum = ssum + rb[h, col]
            v = jnp.maximum(wj * (ssum * inv_h), 0.0)
            vrow[col] = v
            return (jnp.minimum(mn, v), jnp.maximum(mx, v))

        mn16, mx16 = lax.fori_loop(
            0, nchunk, pass1,
            (jnp.full((_LN,), jnp.inf, jnp.float32),
             jnp.full((_LN,), -jnp.inf, jnp.float32)))
        mn, mx = _splat_minmax(mn16, mx16, lbuf)
        scale = 1.0 / jnp.clip(mx - mn, 1e-12, 1.0)

        def pass2(i, carry):
            col = pl.ds(i * _LN, _LN)
            acc[col] = jnp.maximum(acc[col], (vrow[col] - mn) * scale)
            return carry
        lax.fori_loop(0, nchunk, pass2, 0)
        cur = nxt

    pltpu.sync_copy(acc, shared.at[s])
    plsc.subcore_barrier()

    W = S // _NS
    pltpu.sync_copy(shared.at[:, pl.ds(s * W, W)], mbuf)
    for i in range(W // _LN):
        col = pl.ds(i * _LN, _LN)
        m = mbuf[0, col]
        for r in range(1, _NS):
            m = jnp.maximum(m, mbuf[r, col])
        mrow[col] = m
    pltpu.sync_copy(mrow, out_hbm.at[c, pl.ds(s * W, W)])


def _sc_total(attn_flat, rows_hbm, w_hbm, B, H, S, PAIRS):
    mesh = plsc.VectorSubcoreMesh(core_axis_name="c", subcore_axis_name="s",
                                  num_cores=_NC, num_subcores=_NS)
    W = S // _NS
    kfn = functools.partial(
        pl.kernel,
        out_type=jax.ShapeDtypeStruct((B, S), jnp.float32),
        mesh=mesh,
        compiler_params=pltpu.CompilerParams(needs_layout_passes=False),
        scratch_types=[
            pltpu.VMEM((PAIRS, 16), jnp.int32),
            pltpu.VMEM((PAIRS, 16), jnp.float32),
            pltpu.VMEM((H, S), jnp.float32),
            pltpu.VMEM((H, S), jnp.float32),
            pltpu.VMEM((S,), jnp.float32),
            pltpu.VMEM((S,), jnp.float32),
            pltpu.VMEM_SHARED((_NS, S), jnp.float32),
            pltpu.VMEM((_NS, W), jnp.float32),
            pltpu.VMEM((W,), jnp.float32),
            pltpu.VMEM((_LN,), jnp.float32),
            pltpu.SemaphoreType.DMA,
            pltpu.SemaphoreType.DMA,
        ],
    )(functools.partial(_sc_total_body, H, S, PAIRS))
    return kfn(attn_flat, rows_hbm, w_hbm)


def kernel(fore_map, fore_rep_encoded, target_embed, align_attns):
    Lx, B, H, T, S = align_attns.shape
    D = target_embed.shape[-1]
    k = int(0.1 * T)
    PAIRS = -(-k // _NS)
    kp = PAIRS * _NS
    kpad = -(-kp // 128) * 128

    fm = jnp.squeeze(fore_map, axis=1)

    idx_pad, ws_pad = _select(target_embed, fore_rep_encoded, k, kpad)
    idxs = idx_pad[:, :k]

    idx_rs = idx_pad[:, :kp].reshape(B, _NS, PAIRS)
    ws_rs = ws_pad[:, :kp].reshape(B, _NS, PAIRS)
    hh = jnp.arange(H, dtype=jnp.int32)
    bb = jnp.arange(B, dtype=jnp.int32)
    rows = (bb[:, None, None, None] * (H * T)
            + hh[None, None, None, :] * T
            + idx_rs[:, :, :, None])
    pad = jnp.broadcast_to(rows[..., :1], (B, _NS, PAIRS, 16 - H))
    rows16 = jnp.concatenate([rows, pad], axis=-1)
    rows_hbm = rows16.reshape(B * _NS, PAIRS, 16)
    w_hbm = jnp.broadcast_to(ws_rs[..., None],
                             (B, _NS, PAIRS, 16)).reshape(B * _NS, PAIRS, 16)

    attn_flat = align_attns[0].reshape(B * H * T, S)
    total = _sc_total(attn_flat, rows_hbm, w_hbm, B, H, S, PAIRS)
    return (fm, total, idxs)


# speedup vs baseline: 2.3641x; 1.0330x over previous
import functools

import jax
import jax.numpy as jnp
import numpy as np
from jax import lax
from jax.experimental import pallas as pl
from jax.experimental.pallas import tpu as pltpu
from jax.experimental.pallas import tpu_sc as plsc

_EPS = 1e-8
_NC = 2
_NS = 16
_LN = 16


def _select_body(k, kpad, te_ref, fr_ref, idx_ref, ws_ref):
    B, T, D = te_ref.shape
    for b in range(B):
        te = te_ref[b]
        fr = fr_ref[b]
        fr2 = fr[None, :]
        dot = jnp.sum(te * fr2, axis=1)
        n1 = jnp.maximum(jnp.sqrt(jnp.sum(te * te, axis=1)), _EPS)
        n2 = jnp.maximum(jnp.sqrt(jnp.sum(fr * fr)), _EPS)
        w = dot / (n1 * n2)
        wrow = w[None, :]
        wcol = w[:, None]

        C = 128
        rank = jnp.zeros((T,), jnp.int32)
        tt = lax.broadcasted_iota(jnp.int32, (C, T), 1)
        for jc in range(T // C):
            wc = wcol[jc * C:(jc + 1) * C]
            gt = (wc > wrow).astype(jnp.int32)
            jj = lax.broadcasted_iota(jnp.int32, (C, T), 0) + jc * C
            eq = ((wc == wrow) & (jj < tt)).astype(jnp.int32)
            rank = rank + jnp.sum(gt + eq, axis=0)

        rrow = rank[None, :]
        pcol = lax.broadcasted_iota(jnp.int32, (kpad, T), 0)
        tcand = lax.broadcasted_iota(jnp.int32, (kpad, T), 1)
        sel = (rrow == pcol) & (pcol < k)
        idx_ref[b, :] = jnp.sum(jnp.where(sel, tcand, 0), axis=1)
        ws_ref[b, :] = jnp.sum(jnp.where(sel, jnp.broadcast_to(wrow, (kpad, T)),
                                         0.0), axis=1)


def _select(target_embed, fore_rep, k, kpad):
    B, T, D = target_embed.shape
    return pl.pallas_call(
        functools.partial(_select_body, k, kpad),
        out_shape=(
            jax.ShapeDtypeStruct((B, kpad), jnp.int32),
            jax.ShapeDtypeStruct((B, kpad), jnp.float32),
        ),
    )(target_embed, fore_rep)


def _splat_minmax(x_mn, x_mx, buf):
    lanes = lax.iota(jnp.int32, 16)
    for shift in (8, 4, 2, 1):
        perm = lanes ^ shift
        buf[:] = x_mn
        x_mn = jnp.minimum(x_mn, plsc.load_gather(buf, [perm]))
        buf[:] = x_mx
        x_mx = jnp.maximum(x_mx, plsc.load_gather(buf, [perm]))
    return x_mn, x_mx


def _sc_total_body(H, S, PAIRS, attn_hbm, rows_hbm, w_hbm, out_hbm,
                   idx_v, wv, rbuf0, rbuf1, vrow, acc, shared, mbuf, mrow,
                   lbuf, sem0, sem1):
    c = lax.axis_index("c")
    s = lax.axis_index("s")
    wid = c * _NS + s

    pltpu.sync_copy(rows_hbm.at[wid], idx_v)
    pltpu.sync_copy(w_hbm.at[wid], wv)

    nchunk = S // _LN
    U = 4

    zero = jnp.zeros((_LN,), jnp.float32)
    def zstep(i, carry):
        acc[pl.ds(i * _LN, _LN)] = zero
        return carry
    lax.fori_loop(0, nchunk, zstep, 0)

    rbufs = (rbuf0, rbuf1)
    sems = (sem0, sem1)

    def start(j):
        buf = j % 2
        return pltpu.async_copy(attn_hbm.at[idx_v.at[j, pl.ds(0, H)]],
                                rbufs[buf], sems[buf])

    inv_h = jnp.float32(1.0 / H)
    cur = start(0)
    for j in range(PAIRS):
        nxt = start(j + 1) if j + 1 < PAIRS else None
        cur.wait()
        rb = rbufs[j % 2]
        wj = wv[j, :]

        def pass1(i, carry):
            mn, mx = carry
            for u in range(U):
                col = pl.ds((i * U + u) * _LN, _LN)
                ssum = rb[0, col]
                for h in range(1, H):
                    ssum = ssum + rb[h, col]
                v = jnp.maximum(wj * (ssum * inv_h), 0.0)
                vrow[col] = v
                mn = jnp.minimum(mn, v)
                mx = jnp.maximum(mx, v)
            return (mn, mx)

        mn16, mx16 = lax.fori_loop(
            0, nchunk // U, pass1,
            (jnp.full((_LN,), jnp.inf, jnp.float32),
             jnp.full((_LN,), -jnp.inf, jnp.float32)))
        mn, mx = _splat_minmax(mn16, mx16, lbuf)
        scale = 1.0 / jnp.clip(mx - mn, 1e-12, 1.0)

        def pass2(i, carry):
            for u in range(U):
                col = pl.ds((i * U + u) * _LN, _LN)
                acc[col] = jnp.maximum(acc[col], (vrow[col] - mn) * scale)
            return carry
        lax.fori_loop(0, nchunk // U, pass2, 0)
        cur = nxt

    pltpu.sync_copy(acc, shared.at[s])
    plsc.subcore_barrier()

    W = S // _NS
    pltpu.sync_copy(shared.at[:, pl.ds(s * W, W)], mbuf)
    for i in range(W // _LN):
        col = pl.ds(i * _LN, _LN)
        m = mbuf[0, col]
        for r in range(1, _NS):
            m = jnp.maximum(m, mbuf[r, col])
        mrow[col] = m
    pltpu.sync_copy(mrow, out_hbm.at[c, pl.ds(s * W, W)])


def _sc_total(attn_flat, rows_hbm, w_hbm, B, H, S, PAIRS):
    mesh = plsc.VectorSubcoreMesh(core_axis_name="c", subcore_axis_name="s",
                                  num_cores=_NC, num_subcores=_NS)
    W = S // _NS
    kfn = functools.partial(
        pl.kernel,
        out_type=jax.ShapeDtypeStruct((B, S), jnp.float32),
        mesh=mesh,
        compiler_params=pltpu.CompilerParams(needs_layout_passes=False),
        scratch_types=[
            pltpu.VMEM((PAIRS, 16), jnp.int32),
            pltpu.VMEM((PAIRS, 16), jnp.float32),
            pltpu.VMEM((H, S), jnp.float32),
            pltpu.VMEM((H, S), jnp.float32),
            pltpu.VMEM((S,), jnp.float32),
            pltpu.VMEM((S,), jnp.float32),
            pltpu.VMEM_SHARED((_NS, S), jnp.float32),
            pltpu.VMEM((_NS, W), jnp.float32),
            pltpu.VMEM((W,), jnp.float32),
            pltpu.VMEM((_LN,), jnp.float32),
            pltpu.SemaphoreType.DMA,
            pltpu.SemaphoreType.DMA,
        ],
    )(functools.partial(_sc_total_body, H, S, PAIRS))
    return kfn(attn_flat, rows_hbm, w_hbm)


def kernel(fore_map, fore_rep_encoded, target_embed, align_attns):
    Lx, B, H, T, S = align_attns.shape
    k = int(0.1 * T)
    PAIRS = -(-k // _NS)
    kp = PAIRS * _NS
    kpad = -(-kp // 128) * 128

    fm = jnp.squeeze(fore_map, axis=1)

    idx_pad, ws_pad = _select(target_embed, fore_rep_encoded, k, kpad)
    idxs = idx_pad[:, :k]

    hoff = np.zeros((_LN,), np.int32)
    hoff[:H] = np.arange(H, dtype=np.int32) * T
    off = (np.arange(B, dtype=np.int32)[:, None, None, None] * (H * T)
           + hoff[None, None, None, :])
    rows_hbm = (idx_pad[:, :kp].reshape(B, _NS, PAIRS, 1)
                + jnp.asarray(off)).reshape(B * _NS, PAIRS, _LN)
    w_hbm = jnp.broadcast_to(ws_pad[:, :kp].reshape(B, _NS, PAIRS, 1),
                             (B, _NS, PAIRS, _LN)).reshape(B * _NS, PAIRS, _LN)

    attn_flat = align_attns[0].reshape(B * H * T, S)
    total = _sc_total(attn_flat, rows_hbm, w_hbm, B, H, S, PAIRS)
    return (fm, total, idxs)
